# baseline (speedup 1.0000x reference)
import functools

import jax
import jax.numpy as jnp
from jax import lax
from jax.experimental import pallas as pl
from jax.experimental.pallas import tpu as pltpu
from jax.experimental.pallas import tpu_sc as plsc

_NTOK = 8192
_D = 1024
_HALF = 512
_K = 1024
_TILE = 2048
_NSPLIT = 1
_HTOK = _NTOK // _NSPLIT
_NB = _HTOK // _TILE

_NC = 2
_NS = 16
_NW = _NC * _NS
_RPW = _HTOK // _NW
_CHUNK = 32
_NCHUNK = _RPW // _CHUNK


def _bf16_dot(a, b):
    return jnp.dot(a.astype(jnp.bfloat16), b.astype(jnp.bfloat16),
                   preferred_element_type=jnp.float32)


def _rne_bf16_bits(f):
    bits = lax.bitcast_convert_type(f, jnp.uint32)
    return (bits + jnp.uint32(0x7FFF)
            + ((bits >> jnp.uint32(16)) & jnp.uint32(1))) >> jnp.uint32(16)


def _prep_body(cb_c_ref, cb_s_ref, ncb_c_ref, ncb_s_ref, w_comb_ref,
               cbn_c_ref, cbn_s_ref, t_c_ref, t_s_ref):
    cbn_c = cb_c_ref[...] / jnp.maximum(ncb_c_ref[...], 1e-12)
    cbn_s = cb_s_ref[...] / jnp.maximum(ncb_s_ref[...], 1e-12)
    cbn_c_ref[...] = cbn_c.astype(jnp.bfloat16)
    cbn_s_ref[...] = cbn_s.astype(jnp.bfloat16)
    t_c = _bf16_dot(cbn_c, w_comb_ref[0:_HALF, :])
    t_s = _bf16_dot(cbn_s, w_comb_ref[_HALF:, :])
    for t, ref in ((t_c, t_c_ref), (t_s, t_s_ref)):
        lo = _rne_bf16_bits(t[:, :_HALF])
        hi = _rne_bf16_bits(t[:, _HALF:])
        ref[...] = lax.bitcast_convert_type(lo | (hi << jnp.uint32(16)),
                                            jnp.int32)


def _a1_body(x_ref, mu_ref, var_ref,
             lncg_ref, lncb_ref, wc_ref, bc_ref,
             lnsg_ref, lnsb_ref, ws_ref, bs_ref,
             c_ref, s_ref):
    xx = x_ref[...]
    xn = (xx - mu_ref[...]) / jnp.sqrt(var_ref[...] + 1e-5)
    a_c = xn * lncg_ref[...] + lncb_ref[...]
    c_ref[...] = jnp.tanh(_bf16_dot(a_c, wc_ref[...]) + bc_ref[...])
    a_s = xn * lnsg_ref[...] + lnsb_ref[...]
    s_ref[...] = jnp.tanh(_bf16_dot(a_s, ws_ref[...]) + bs_ref[...])


def _a2_body(c_ref, s_ref, nc_ref, ns_ref, cbnc_ref, cbns_ref,
             cidx_ref, sidx_ref, acc_ref):
    i = pl.program_id(0)
    content = c_ref[...]
    style = s_ref[...]
    nc = jnp.maximum(nc_ref[...], 1e-12)
    ns = jnp.maximum(ns_ref[...], 1e-12)
    cn = content / nc
    sn = style / ns

    d_c = 1.0 - lax.dot_general(cn.astype(jnp.bfloat16), cbnc_ref[...],
                                (((1,), (1,)), ((), ())),
                                preferred_element_type=jnp.float32)
    d_s = 1.0 - lax.dot_general(sn.astype(jnp.bfloat16), cbns_ref[...],
                                (((1,), (1,)), ((), ())),
                                preferred_element_type=jnp.float32)
    dcmin = jnp.min(d_c, axis=1, keepdims=True)
    dsmin = jnp.min(d_s, axis=1, keepdims=True)
    ii = lax.broadcasted_iota(jnp.int32, d_c.shape, 1)
    big = jnp.int32(2 ** 30)
    cidx_ref[0, 0, :] = jnp.min(jnp.where(d_c == dcmin, ii, big), axis=1)
    sidx_ref[0, 0, :] = jnp.min(jnp.where(d_s == dsmin, ii, big), axis=1)

    ncf = nc[:, 0]
    nsf = ns[:, 0]
    e_c = jnp.sum(1.0 + ncf * ncf - 2.0 * ncf * (1.0 - dcmin[:, 0]))
    e_s = jnp.sum(1.0 + nsf * nsf - 2.0 * nsf * (1.0 - dsmin[:, 0]))
    cosv = jnp.sum(jnp.abs(jnp.sum(cn * sn, axis=1)))

    row = lax.broadcasted_iota(jnp.int32, (8, 128), 0)
    col = lax.broadcasted_iota(jnp.int32, (8, 128), 1)
    vals = (jnp.where((row == 0) & (col == 0), e_c, 0.0)
            + jnp.where((row == 1) & (col == 0), e_s, 0.0)
            + jnp.where((row == 2) & (col == 0), cosv, 0.0))

    @pl.when(i == 0)
    def _():
        acc_ref[...] = jnp.zeros((8, 128), jnp.float32)

    acc_ref[...] += vals


def _gather_body(tc_hbm, ts_hbm, cidx_hbm, sidx_hbm,
                 y1_hbm, y2_hbm,
                 ci_v, si_v, bufc, bufs, sem1, sem2):
    wid = lax.axis_index("s") * _NC + lax.axis_index("c")
    base = wid * _RPW
    pltpu.sync_copy(cidx_hbm.at[pl.ds(base, _RPW)], ci_v)
    pltpu.sync_copy(sidx_hbm.at[pl.ds(base, _RPW)], si_v)

    def body(k, carry):
        r0 = k * _CHUNK
        cpc = pltpu.async_copy(tc_hbm.at[ci_v.at[pl.ds(r0, _CHUNK)]], bufc,
                               sem1)
        cps = pltpu.async_copy(ts_hbm.at[si_v.at[pl.ds(r0, _CHUNK)]], bufs,
                               sem2)
        cpc.wait()
        pltpu.sync_copy(bufc, y1_hbm.at[pl.ds(base + r0, _CHUNK)])
        cps.wait()
        pltpu.sync_copy(bufs, y2_hbm.at[pl.ds(base + r0, _CHUNK)])
        return carry

    lax.fori_loop(0, _NCHUNK, body, 0)


def _unpack_pair(w):
    lo = lax.bitcast_convert_type(jnp.left_shift(w, 16), jnp.float32)
    hi = lax.bitcast_convert_type(jnp.bitwise_and(w, jnp.int32(-65536)),
                                  jnp.float32)
    return lo, hi


def _ln_body(y1_ref, y2_ref, bcomb_ref, g_ref, b_ref, out_ref):
    lo1, hi1 = _unpack_pair(y1_ref[...])
    lo2, hi2 = _unpack_pair(y2_ref[...])
    y = (jnp.concatenate([lo1 + lo2, hi1 + hi2], axis=1) + bcomb_ref[...])
    mu = jnp.mean(y, axis=1, keepdims=True)
    var = jnp.mean((y - mu) ** 2, axis=1, keepdims=True)
    out_ref[...] = (y - mu) / jnp.sqrt(var + 1e-5) * g_ref[...] + b_ref[...]


def kernel(x, ln_c_g, ln_c_b, W_c, b_c, ln_s_g, ln_s_b, W_s, b_s,
           cb_c, cb_s, W_comb, b_comb, ln_o_g, ln_o_b):
    B, S, D = x.shape
    x2d = x.reshape(B * S, D)

    mu = jnp.mean(x2d, axis=-1, keepdims=True)
    var = jnp.mean((x2d - mu) ** 2, axis=-1, keepdims=True)
    ncb_c = jnp.linalg.norm(cb_c, axis=-1, keepdims=True)
    ncb_s = jnp.linalg.norm(cb_s, axis=-1, keepdims=True)

    cbn_c, cbn_s, t_c_pk, t_s_pk = pl.pallas_call(
        _prep_body,
        out_shape=[
            jax.ShapeDtypeStruct((_K, _HALF), jnp.bfloat16),
            jax.ShapeDtypeStruct((_K, _HALF), jnp.bfloat16),
            jax.ShapeDtypeStruct((_K, _HALF), jnp.int32),
            jax.ShapeDtypeStruct((_K, _HALF), jnp.int32),
        ],
    )(cb_c, cb_s, ncb_c, ncb_s, W_comb)

    full = lambda shape: pl.BlockSpec(shape, lambda i: (0,) * len(shape))
    col = lambda: pl.BlockSpec((_TILE, 1), lambda i: (i, 0))
    row_d = lambda: pl.BlockSpec((_TILE, _D), lambda i: (i, 0))
    row_h = lambda: pl.BlockSpec((_TILE, _HALF), lambda i: (i, 0))

    a1_call = pl.pallas_call(
        _a1_body,
        grid=(_NB,),
        in_specs=[
            row_d(), col(), col(),
            full((1, _D)), full((1, _D)), full((_D, _HALF)), full((1, _HALF)),
            full((1, _D)), full((1, _D)), full((_D, _HALF)), full((1, _HALF)),
        ],
        out_specs=[row_h(), row_h()],
        out_shape=[
            jax.ShapeDtypeStruct((_HTOK, _HALF), jnp.float32),
            jax.ShapeDtypeStruct((_HTOK, _HALF), jnp.float32),
        ],
    )

    a2_call = pl.pallas_call(
        _a2_body,
        grid=(_NB,),
        in_specs=[
            row_h(), row_h(), col(), col(),
            full((_K, _HALF)), full((_K, _HALF)),
        ],
        out_specs=[
            pl.BlockSpec((1, 1, _TILE), lambda i: (i, 0, 0)),
            pl.BlockSpec((1, 1, _TILE), lambda i: (i, 0, 0)),
            pl.BlockSpec((8, 128), lambda i: (0, 0)),
        ],
        out_shape=[
            jax.ShapeDtypeStruct((_NB, 1, _TILE), jnp.int32),
            jax.ShapeDtypeStruct((_NB, 1, _TILE), jnp.int32),
            jax.ShapeDtypeStruct((8, 128), jnp.float32),
        ],
    )

    sc_gather = functools.partial(
        pl.kernel,
        mesh=plsc.VectorSubcoreMesh(core_axis_name="c", subcore_axis_name="s"),
        out_type=[
            jax.ShapeDtypeStruct((_HTOK, _HALF), jnp.int32),
            jax.ShapeDtypeStruct((_HTOK, _HALF), jnp.int32),
        ],
        scratch_types=[
            pltpu.VMEM((_RPW,), jnp.int32),
            pltpu.VMEM((_RPW,), jnp.int32),
            pltpu.VMEM((_CHUNK, _HALF), jnp.int32),
            pltpu.VMEM((_CHUNK, _HALF), jnp.int32),
            pltpu.SemaphoreType.DMA,
            pltpu.SemaphoreType.DMA,
        ],
    )(_gather_body)

    ln_call = pl.pallas_call(
        _ln_body,
        grid=(_NB,),
        in_specs=[row_h(), row_h(), full((1, _D)), full((1, _D)),
                  full((1, _D))],
        out_specs=row_d(),
        out_shape=jax.ShapeDtypeStruct((_HTOK, _D), jnp.float32),
    )

    ln_vec_args = (ln_c_g.reshape(1, _D), ln_c_b.reshape(1, _D), W_c,
                   b_c.reshape(1, _HALF),
                   ln_s_g.reshape(1, _D), ln_s_b.reshape(1, _D), W_s,
                   b_s.reshape(1, _HALF))

    cidx_parts, sidx_parts, acc_parts, y_parts = [], [], [], []
    for h in range(_NSPLIT):
        lo, hi = h * _HTOK, (h + 1) * _HTOK
        xh = lax.slice_in_dim(x2d, lo, hi, axis=0)
        muh = lax.slice_in_dim(mu, lo, hi, axis=0)
        varh = lax.slice_in_dim(var, lo, hi, axis=0)
        content_h, style_h = a1_call(xh, muh, varh, *ln_vec_args)
        n_c_h = jnp.linalg.norm(content_h, axis=-1, keepdims=True)
        n_s_h = jnp.linalg.norm(style_h, axis=-1, keepdims=True)
        cidx3, sidx3, acc_h = a2_call(content_h, style_h, n_c_h, n_s_h,
                                      cbn_c, cbn_s)
        ci_h = cidx3.reshape(_HTOK)
        si_h = sidx3.reshape(_HTOK)
        cidx_parts.append(ci_h)
        sidx_parts.append(si_h)
        acc_parts.append(acc_h)
        y_parts.append(sc_gather(t_c_pk, t_s_pk, ci_h, si_h))

    comb_parts = [
        ln_call(y1, y2, b_comb.reshape(1, _D), ln_o_g.reshape(1, _D),
                ln_o_b.reshape(1, _D))
        for (y1, y2) in y_parts
    ]
    combined2d = jnp.concatenate(comb_parts, axis=0)
    cidx_flat = jnp.concatenate(cidx_parts)
    sidx_flat = jnp.concatenate(sidx_parts)
    acc = acc_parts[0]
    for a in acc_parts[1:]:
        acc = acc + a

    e_c = acc[0, 0]
    e_s = acc[1, 0]
    cos_sum = acc[2, 0]
    closs = 0.1 * (e_c / (_NTOK * _HALF))
    sloss = 0.1 * (e_s / (_NTOK * _HALF))
    disentangle_loss = jnp.clip(cos_sum / _NTOK, 0.0, 1.0)
    total_loss = closs + sloss + 0.5 * disentangle_loss

    combined = combined2d.reshape(B, S, D)
    cidx = cidx_flat.reshape(B, S)
    sidx = sidx_flat.reshape(B, S)
    return combined, total_loss, cidx, sidx, disentangle_loss


# speedup vs baseline: 1.0151x; 1.0151x over previous
import functools

import jax
import jax.numpy as jnp
from jax import lax
from jax.experimental import pallas as pl
from jax.experimental.pallas import tpu as pltpu
from jax.experimental.pallas import tpu_sc as plsc

_NTOK = 8192
_D = 1024
_HALF = 512
_K = 1024
_TILE = 1024
_NSPLIT = 1
_HTOK = _NTOK // _NSPLIT
_NB = _HTOK // _TILE

_NC = 2
_NS = 16
_NW = _NC * _NS
_RPW = _HTOK // _NW
_CHUNK = 32
_NCHUNK = _RPW // _CHUNK


def _bf16_dot(a, b):
    return jnp.dot(a.astype(jnp.bfloat16), b.astype(jnp.bfloat16),
                   preferred_element_type=jnp.float32)


def _rne_bf16_bits(f):
    bits = lax.bitcast_convert_type(f, jnp.uint32)
    return (bits + jnp.uint32(0x7FFF)
            + ((bits >> jnp.uint32(16)) & jnp.uint32(1))) >> jnp.uint32(16)


def _prep_body(cb_c_ref, cb_s_ref, ncb_c_ref, ncb_s_ref, w_comb_ref,
               cbn_c_ref, cbn_s_ref, t_c_ref, t_s_ref):
    cbn_c = cb_c_ref[...] / jnp.maximum(ncb_c_ref[...], 1e-12)
    cbn_s = cb_s_ref[...] / jnp.maximum(ncb_s_ref[...], 1e-12)
    cbn_c_ref[...] = cbn_c.astype(jnp.bfloat16)
    cbn_s_ref[...] = cbn_s.astype(jnp.bfloat16)
    t_c = _bf16_dot(cbn_c, w_comb_ref[0:_HALF, :])
    t_s = _bf16_dot(cbn_s, w_comb_ref[_HALF:, :])
    for t, ref in ((t_c, t_c_ref), (t_s, t_s_ref)):
        lo = _rne_bf16_bits(t[:, :_HALF])
        hi = _rne_bf16_bits(t[:, _HALF:])
        ref[...] = lax.bitcast_convert_type(lo | (hi << jnp.uint32(16)),
                                            jnp.int32)


def _a1_body(x_ref, mu_ref, var_ref,
             lncg_ref, lncb_ref, wc_ref, bc_ref,
             lnsg_ref, lnsb_ref, ws_ref, bs_ref,
             c_ref, s_ref):
    xx = x_ref[...]
    xn = (xx - mu_ref[...]) / jnp.sqrt(var_ref[...] + 1e-5)
    a_c = xn * lncg_ref[...] + lncb_ref[...]
    c_ref[...] = jnp.tanh(_bf16_dot(a_c, wc_ref[...]) + bc_ref[...])
    a_s = xn * lnsg_ref[...] + lnsb_ref[...]
    s_ref[...] = jnp.tanh(_bf16_dot(a_s, ws_ref[...]) + bs_ref[...])


def _a2_body(c_ref, s_ref, nc_ref, ns_ref, cbnc_ref, cbns_ref,
             cidx_ref, sidx_ref, acc_ref):
    i = pl.program_id(0)
    content = c_ref[...]
    style = s_ref[...]
    nc = jnp.maximum(nc_ref[...], 1e-12)
    ns = jnp.maximum(ns_ref[...], 1e-12)
    cn = content / nc
    sn = style / ns

    d_c = 1.0 - lax.dot_general(cn.astype(jnp.bfloat16), cbnc_ref[...],
                                (((1,), (1,)), ((), ())),
                                preferred_element_type=jnp.float32)
    d_s = 1.0 - lax.dot_general(sn.astype(jnp.bfloat16), cbns_ref[...],
                                (((1,), (1,)), ((), ())),
                                preferred_element_type=jnp.float32)
    dcmin = jnp.min(d_c, axis=1, keepdims=True)
    dsmin = jnp.min(d_s, axis=1, keepdims=True)
    ii = lax.broadcasted_iota(jnp.int32, d_c.shape, 1)
    big = jnp.int32(2 ** 30)
    cidx_ref[0, 0, :] = jnp.min(jnp.where(d_c == dcmin, ii, big), axis=1)
    sidx_ref[0, 0, :] = jnp.min(jnp.where(d_s == dsmin, ii, big), axis=1)

    ncf = nc[:, 0]
    nsf = ns[:, 0]
    e_c = jnp.sum(1.0 + ncf * ncf - 2.0 * ncf * (1.0 - dcmin[:, 0]))
    e_s = jnp.sum(1.0 + nsf * nsf - 2.0 * nsf * (1.0 - dsmin[:, 0]))
    cosv = jnp.sum(jnp.abs(jnp.sum(cn * sn, axis=1)))

    row = lax.broadcasted_iota(jnp.int32, (8, 128), 0)
    col = lax.broadcasted_iota(jnp.int32, (8, 128), 1)
    vals = (jnp.where((row == 0) & (col == 0), e_c, 0.0)
            + jnp.where((row == 1) & (col == 0), e_s, 0.0)
            + jnp.where((row == 2) & (col == 0), cosv, 0.0))

    @pl.when(i == 0)
    def _():
        acc_ref[...] = jnp.zeros((8, 128), jnp.float32)

    acc_ref[...] += vals


def _gather_body(tc_hbm, ts_hbm, cidx_hbm, sidx_hbm,
                 y1_hbm, y2_hbm,
                 ci_v, si_v, bufc, bufs, sem1, sem2):
    wid = lax.axis_index("s") * _NC + lax.axis_index("c")
    base = wid * _RPW
    pltpu.sync_copy(cidx_hbm.at[pl.ds(base, _RPW)], ci_v)
    pltpu.sync_copy(sidx_hbm.at[pl.ds(base, _RPW)], si_v)

    def body(k, carry):
        r0 = k * _CHUNK
        cpc = pltpu.async_copy(tc_hbm.at[ci_v.at[pl.ds(r0, _CHUNK)]], bufc,
                               sem1)
        cps = pltpu.async_copy(ts_hbm.at[si_v.at[pl.ds(r0, _CHUNK)]], bufs,
                               sem2)
        cpc.wait()
        pltpu.sync_copy(bufc, y1_hbm.at[pl.ds(base + r0, _CHUNK)])
        cps.wait()
        pltpu.sync_copy(bufs, y2_hbm.at[pl.ds(base + r0, _CHUNK)])
        return carry

    lax.fori_loop(0, _NCHUNK, body, 0)


def _unpack_pair(w):
    lo = lax.bitcast_convert_type(jnp.left_shift(w, 16), jnp.float32)
    hi = lax.bitcast_convert_type(jnp.bitwise_and(w, jnp.int32(-65536)),
                                  jnp.float32)
    return lo, hi


def _ln_body(y1_ref, y2_ref, bcomb_ref, g_ref, b_ref, out_ref):
    lo1, hi1 = _unpack_pair(y1_ref[...])
    lo2, hi2 = _unpack_pair(y2_ref[...])
    y = (jnp.concatenate([lo1 + lo2, hi1 + hi2], axis=1) + bcomb_ref[...])
    mu = jnp.mean(y, axis=1, keepdims=True)
    var = jnp.mean((y - mu) ** 2, axis=1, keepdims=True)
    out_ref[...] = (y - mu) / jnp.sqrt(var + 1e-5) * g_ref[...] + b_ref[...]


def kernel(x, ln_c_g, ln_c_b, W_c, b_c, ln_s_g, ln_s_b, W_s, b_s,
           cb_c, cb_s, W_comb, b_comb, ln_o_g, ln_o_b):
    B, S, D = x.shape
    x2d = x.reshape(B * S, D)

    mu = jnp.mean(x2d, axis=-1, keepdims=True)
    var = jnp.mean((x2d - mu) ** 2, axis=-1, keepdims=True)
    ncb_c = jnp.linalg.norm(cb_c, axis=-1, keepdims=True)
    ncb_s = jnp.linalg.norm(cb_s, axis=-1, keepdims=True)

    cbn_c, cbn_s, t_c_pk, t_s_pk = pl.pallas_call(
        _prep_body,
        out_shape=[
            jax.ShapeDtypeStruct((_K, _HALF), jnp.bfloat16),
            jax.ShapeDtypeStruct((_K, _HALF), jnp.bfloat16),
            jax.ShapeDtypeStruct((_K, _HALF), jnp.int32),
            jax.ShapeDtypeStruct((_K, _HALF), jnp.int32),
        ],
    )(cb_c, cb_s, ncb_c, ncb_s, W_comb)

    full = lambda shape: pl.BlockSpec(shape, lambda i: (0,) * len(shape))
    col = lambda: pl.BlockSpec((_TILE, 1), lambda i: (i, 0))
    row_d = lambda: pl.BlockSpec((_TILE, _D), lambda i: (i, 0))
    row_h = lambda: pl.BlockSpec((_TILE, _HALF), lambda i: (i, 0))

    a1_call = pl.pallas_call(
        _a1_body,
        grid=(_NB,),
        in_specs=[
            row_d(), col(), col(),
            full((1, _D)), full((1, _D)), full((_D, _HALF)), full((1, _HALF)),
            full((1, _D)), full((1, _D)), full((_D, _HALF)), full((1, _HALF)),
        ],
        out_specs=[row_h(), row_h()],
        out_shape=[
            jax.ShapeDtypeStruct((_HTOK, _HALF), jnp.float32),
            jax.ShapeDtypeStruct((_HTOK, _HALF), jnp.float32),
        ],
    )

    a2_call = pl.pallas_call(
        _a2_body,
        grid=(_NB,),
        in_specs=[
            row_h(), row_h(), col(), col(),
            full((_K, _HALF)), full((_K, _HALF)),
        ],
        out_specs=[
            pl.BlockSpec((1, 1, _TILE), lambda i: (i, 0, 0)),
            pl.BlockSpec((1, 1, _TILE), lambda i: (i, 0, 0)),
            pl.BlockSpec((8, 128), lambda i: (0, 0)),
        ],
        out_shape=[
            jax.ShapeDtypeStruct((_NB, 1, _TILE), jnp.int32),
            jax.ShapeDtypeStruct((_NB, 1, _TILE), jnp.int32),
            jax.ShapeDtypeStruct((8, 128), jnp.float32),
        ],
    )

    sc_gather = functools.partial(
        pl.kernel,
        mesh=plsc.VectorSubcoreMesh(core_axis_name="c", subcore_axis_name="s"),
        out_type=[
            jax.ShapeDtypeStruct((_HTOK, _HALF), jnp.int32),
            jax.ShapeDtypeStruct((_HTOK, _HALF), jnp.int32),
        ],
        scratch_types=[
            pltpu.VMEM((_RPW,), jnp.int32),
            pltpu.VMEM((_RPW,), jnp.int32),
            pltpu.VMEM((_CHUNK, _HALF), jnp.int32),
            pltpu.VMEM((_CHUNK, _HALF), jnp.int32),
            pltpu.SemaphoreType.DMA,
            pltpu.SemaphoreType.DMA,
        ],
    )(_gather_body)

    ln_call = pl.pallas_call(
        _ln_body,
        grid=(_NB,),
        in_specs=[row_h(), row_h(), full((1, _D)), full((1, _D)),
                  full((1, _D))],
        out_specs=row_d(),
        out_shape=jax.ShapeDtypeStruct((_HTOK, _D), jnp.float32),
    )

    ln_vec_args = (ln_c_g.reshape(1, _D), ln_c_b.reshape(1, _D), W_c,
                   b_c.reshape(1, _HALF),
                   ln_s_g.reshape(1, _D), ln_s_b.reshape(1, _D), W_s,
                   b_s.reshape(1, _HALF))

    cidx_parts, sidx_parts, acc_parts, y_parts = [], [], [], []
    for h in range(_NSPLIT):
        lo, hi = h * _HTOK, (h + 1) * _HTOK
        xh = lax.slice_in_dim(x2d, lo, hi, axis=0)
        muh = lax.slice_in_dim(mu, lo, hi, axis=0)
        varh = lax.slice_in_dim(var, lo, hi, axis=0)
        content_h, style_h = a1_call(xh, muh, varh, *ln_vec_args)
        n_c_h = jnp.linalg.norm(content_h, axis=-1, keepdims=True)
        n_s_h = jnp.linalg.norm(style_h, axis=-1, keepdims=True)
        cidx3, sidx3, acc_h = a2_call(content_h, style_h, n_c_h, n_s_h,
                                      cbn_c, cbn_s)
        ci_h = cidx3.reshape(_HTOK)
        si_h = sidx3.reshape(_HTOK)
        cidx_parts.append(ci_h)
        sidx_parts.append(si_h)
        acc_parts.append(acc_h)
        y_parts.append(sc_gather(t_c_pk, t_s_pk, ci_h, si_h))

    comb_parts = [
        ln_call(y1, y2, b_comb.reshape(1, _D), ln_o_g.reshape(1, _D),
                ln_o_b.reshape(1, _D))
        for (y1, y2) in y_parts
    ]
    combined2d = jnp.concatenate(comb_parts, axis=0)
    cidx_flat = jnp.concatenate(cidx_parts)
    sidx_flat = jnp.concatenate(sidx_parts)
    acc = acc_parts[0]
    for a in acc_parts[1:]:
        acc = acc + a

    e_c = acc[0, 0]
    e_s = acc[1, 0]
    cos_sum = acc[2, 0]
    closs = 0.1 * (e_c / (_NTOK * _HALF))
    sloss = 0.1 * (e_s / (_NTOK * _HALF))
    disentangle_loss = jnp.clip(cos_sum / _NTOK, 0.0, 1.0)
    total_loss = closs + sloss + 0.5 * disentangle_loss

    combined = combined2d.reshape(B, S, D)
    cidx = cidx_flat.reshape(B, S)
    sidx = sidx_flat.reshape(B, S)
    return combined, total_loss, cidx, sidx, disentangle_loss


# speedup vs baseline: 1.0210x; 1.0059x over previous
import functools

import jax
import jax.numpy as jnp
from jax import lax
from jax.experimental import pallas as pl
from jax.experimental.pallas import tpu as pltpu
from jax.experimental.pallas import tpu_sc as plsc

_NTOK = 8192
_D = 1024
_HALF = 512
_K = 1024
_TILE = 1024
_NSPLIT = 1
_HTOK = _NTOK // _NSPLIT
_NB = _HTOK // _TILE

_NC = 2
_NS = 16
_NW = _NC * _NS
_RPW = _HTOK // _NW
_CHUNK = 64
_NCHUNK = _RPW // _CHUNK


def _bf16_dot(a, b):
    return jnp.dot(a.astype(jnp.bfloat16), b.astype(jnp.bfloat16),
                   preferred_element_type=jnp.float32)


def _rne_bf16_bits(f):
    bits = lax.bitcast_convert_type(f, jnp.uint32)
    return (bits + jnp.uint32(0x7FFF)
            + ((bits >> jnp.uint32(16)) & jnp.uint32(1))) >> jnp.uint32(16)


def _prep_body(cb_c_ref, cb_s_ref, ncb_c_ref, ncb_s_ref, w_comb_ref,
               cbn_c_ref, cbn_s_ref, t_c_ref, t_s_ref):
    cbn_c = cb_c_ref[...] / jnp.maximum(ncb_c_ref[...], 1e-12)
    cbn_s = cb_s_ref[...] / jnp.maximum(ncb_s_ref[...], 1e-12)
    cbn_c_ref[...] = cbn_c.astype(jnp.bfloat16)
    cbn_s_ref[...] = cbn_s.astype(jnp.bfloat16)
    t_c = _bf16_dot(cbn_c, w_comb_ref[0:_HALF, :])
    t_s = _bf16_dot(cbn_s, w_comb_ref[_HALF:, :])
    for t, ref in ((t_c, t_c_ref), (t_s, t_s_ref)):
        lo = _rne_bf16_bits(t[:, :_HALF])
        hi = _rne_bf16_bits(t[:, _HALF:])
        ref[...] = lax.bitcast_convert_type(lo | (hi << jnp.uint32(16)),
                                            jnp.int32)


def _a1_body(x_ref, mu_ref, var_ref,
             lncg_ref, lncb_ref, wc_ref, bc_ref,
             lnsg_ref, lnsb_ref, ws_ref, bs_ref,
             c_ref, s_ref):
    xx = x_ref[...]
    xn = (xx - mu_ref[...]) / jnp.sqrt(var_ref[...] + 1e-5)
    a_c = xn * lncg_ref[...] + lncb_ref[...]
    c_ref[...] = jnp.tanh(_bf16_dot(a_c, wc_ref[...]) + bc_ref[...])
    a_s = xn * lnsg_ref[...] + lnsb_ref[...]
    s_ref[...] = jnp.tanh(_bf16_dot(a_s, ws_ref[...]) + bs_ref[...])


def _a2_body(c_ref, s_ref, nc_ref, ns_ref, cbnc_ref, cbns_ref,
             cidx_ref, sidx_ref, acc_ref):
    i = pl.program_id(0)
    content = c_ref[...]
    style = s_ref[...]
    nc = jnp.maximum(nc_ref[...], 1e-12)
    ns = jnp.maximum(ns_ref[...], 1e-12)
    cn = content / nc
    sn = style / ns

    d_c = 1.0 - lax.dot_general(cn.astype(jnp.bfloat16), cbnc_ref[...],
                                (((1,), (1,)), ((), ())),
                                preferred_element_type=jnp.float32)
    d_s = 1.0 - lax.dot_general(sn.astype(jnp.bfloat16), cbns_ref[...],
                                (((1,), (1,)), ((), ())),
                                preferred_element_type=jnp.float32)
    dcmin = jnp.min(d_c, axis=1, keepdims=True)
    dsmin = jnp.min(d_s, axis=1, keepdims=True)
    ii = lax.broadcasted_iota(jnp.int32, d_c.shape, 1)
    big = jnp.int32(2 ** 30)
    cidx_ref[0, 0, :] = jnp.min(jnp.where(d_c == dcmin, ii, big), axis=1)
    sidx_ref[0, 0, :] = jnp.min(jnp.where(d_s == dsmin, ii, big), axis=1)

    ncf = nc[:, 0]
    nsf = ns[:, 0]
    e_c = jnp.sum(1.0 + ncf * ncf - 2.0 * ncf * (1.0 - dcmin[:, 0]))
    e_s = jnp.sum(1.0 + nsf * nsf - 2.0 * nsf * (1.0 - dsmin[:, 0]))
    cosv = jnp.sum(jnp.abs(jnp.sum(cn * sn, axis=1)))

    row = lax.broadcasted_iota(jnp.int32, (8, 128), 0)
    col = lax.broadcasted_iota(jnp.int32, (8, 128), 1)
    vals = (jnp.where((row == 0) & (col == 0), e_c, 0.0)
            + jnp.where((row == 1) & (col == 0), e_s, 0.0)
            + jnp.where((row == 2) & (col == 0), cosv, 0.0))

    @pl.when(i == 0)
    def _():
        acc_ref[...] = jnp.zeros((8, 128), jnp.float32)

    acc_ref[...] += vals


def _gather_body(tc_hbm, ts_hbm, cidx_hbm, sidx_hbm,
                 y1_hbm, y2_hbm,
                 ci_v, si_v, bufc, bufs, sem1, sem2):
    wid = lax.axis_index("s") * _NC + lax.axis_index("c")
    base = wid * _RPW
    pltpu.sync_copy(cidx_hbm.at[pl.ds(base, _RPW)], ci_v)
    pltpu.sync_copy(sidx_hbm.at[pl.ds(base, _RPW)], si_v)

    def body(k, carry):
        r0 = k * _CHUNK
        cpc = pltpu.async_copy(tc_hbm.at[ci_v.at[pl.ds(r0, _CHUNK)]], bufc,
                               sem1)
        cps = pltpu.async_copy(ts_hbm.at[si_v.at[pl.ds(r0, _CHUNK)]], bufs,
                               sem2)
        cpc.wait()
        pltpu.sync_copy(bufc, y1_hbm.at[pl.ds(base + r0, _CHUNK)])
        cps.wait()
        pltpu.sync_copy(bufs, y2_hbm.at[pl.ds(base + r0, _CHUNK)])
        return carry

    lax.fori_loop(0, _NCHUNK, body, 0)


def _unpack_pair(w):
    lo = lax.bitcast_convert_type(jnp.left_shift(w, 16), jnp.float32)
    hi = lax.bitcast_convert_type(jnp.bitwise_and(w, jnp.int32(-65536)),
                                  jnp.float32)
    return lo, hi


def _ln_body(y1_ref, y2_ref, bcomb_ref, g_ref, b_ref, out_ref):
    lo1, hi1 = _unpack_pair(y1_ref[...])
    lo2, hi2 = _unpack_pair(y2_ref[...])
    y = (jnp.concatenate([lo1 + lo2, hi1 + hi2], axis=1) + bcomb_ref[...])
    mu = jnp.mean(y, axis=1, keepdims=True)
    var = jnp.mean((y - mu) ** 2, axis=1, keepdims=True)
    out_ref[...] = (y - mu) / jnp.sqrt(var + 1e-5) * g_ref[...] + b_ref[...]


def kernel(x, ln_c_g, ln_c_b, W_c, b_c, ln_s_g, ln_s_b, W_s, b_s,
           cb_c, cb_s, W_comb, b_comb, ln_o_g, ln_o_b):
    B, S, D = x.shape
    x2d = x.reshape(B * S, D)

    mu = jnp.mean(x2d, axis=-1, keepdims=True)
    var = jnp.mean((x2d - mu) ** 2, axis=-1, keepdims=True)
    ncb_c = jnp.linalg.norm(cb_c, axis=-1, keepdims=True)
    ncb_s = jnp.linalg.norm(cb_s, axis=-1, keepdims=True)

    cbn_c, cbn_s, t_c_pk, t_s_pk = pl.pallas_call(
        _prep_body,
        out_shape=[
            jax.ShapeDtypeStruct((_K, _HALF), jnp.bfloat16),
            jax.ShapeDtypeStruct((_K, _HALF), jnp.bfloat16),
            jax.ShapeDtypeStruct((_K, _HALF), jnp.int32),
            jax.ShapeDtypeStruct((_K, _HALF), jnp.int32),
        ],
    )(cb_c, cb_s, ncb_c, ncb_s, W_comb)

    full = lambda shape: pl.BlockSpec(shape, lambda i: (0,) * len(shape))
    col = lambda: pl.BlockSpec((_TILE, 1), lambda i: (i, 0))
    row_d = lambda: pl.BlockSpec((_TILE, _D), lambda i: (i, 0))
    row_h = lambda: pl.BlockSpec((_TILE, _HALF), lambda i: (i, 0))

    a1_call = pl.pallas_call(
        _a1_body,
        grid=(_NB,),
        in_specs=[
            row_d(), col(), col(),
            full((1, _D)), full((1, _D)), full((_D, _HALF)), full((1, _HALF)),
            full((1, _D)), full((1, _D)), full((_D, _HALF)), full((1, _HALF)),
        ],
        out_specs=[row_h(), row_h()],
        out_shape=[
            jax.ShapeDtypeStruct((_HTOK, _HALF), jnp.float32),
            jax.ShapeDtypeStruct((_HTOK, _HALF), jnp.float32),
        ],
    )

    a2_call = pl.pallas_call(
        _a2_body,
        grid=(_NB,),
        in_specs=[
            row_h(), row_h(), col(), col(),
            full((_K, _HALF)), full((_K, _HALF)),
        ],
        out_specs=[
            pl.BlockSpec((1, 1, _TILE), lambda i: (i, 0, 0)),
            pl.BlockSpec((1, 1, _TILE), lambda i: (i, 0, 0)),
            pl.BlockSpec((8, 128), lambda i: (0, 0)),
        ],
        out_shape=[
            jax.ShapeDtypeStruct((_NB, 1, _TILE), jnp.int32),
            jax.ShapeDtypeStruct((_NB, 1, _TILE), jnp.int32),
            jax.ShapeDtypeStruct((8, 128), jnp.float32),
        ],
    )

    sc_gather = functools.partial(
        pl.kernel,
        mesh=plsc.VectorSubcoreMesh(core_axis_name="c", subcore_axis_name="s"),
        out_type=[
            jax.ShapeDtypeStruct((_HTOK, _HALF), jnp.int32),
            jax.ShapeDtypeStruct((_HTOK, _HALF), jnp.int32),
        ],
        scratch_types=[
            pltpu.VMEM((_RPW,), jnp.int32),
            pltpu.VMEM((_RPW,), jnp.int32),
            pltpu.VMEM((_CHUNK, _HALF), jnp.int32),
            pltpu.VMEM((_CHUNK, _HALF), jnp.int32),
            pltpu.SemaphoreType.DMA,
            pltpu.SemaphoreType.DMA,
        ],
    )(_gather_body)

    ln_call = pl.pallas_call(
        _ln_body,
        grid=(_NB,),
        in_specs=[row_h(), row_h(), full((1, _D)), full((1, _D)),
                  full((1, _D))],
        out_specs=row_d(),
        out_shape=jax.ShapeDtypeStruct((_HTOK, _D), jnp.float32),
    )

    ln_vec_args = (ln_c_g.reshape(1, _D), ln_c_b.reshape(1, _D), W_c,
                   b_c.reshape(1, _HALF),
                   ln_s_g.reshape(1, _D), ln_s_b.reshape(1, _D), W_s,
                   b_s.reshape(1, _HALF))

    cidx_parts, sidx_parts, acc_parts, y_parts = [], [], [], []
    for h in range(_NSPLIT):
        lo, hi = h * _HTOK, (h + 1) * _HTOK
        xh = lax.slice_in_dim(x2d, lo, hi, axis=0)
        muh = lax.slice_in_dim(mu, lo, hi, axis=0)
        varh = lax.slice_in_dim(var, lo, hi, axis=0)
        content_h, style_h = a1_call(xh, muh, varh, *ln_vec_args)
        n_c_h = jnp.linalg.norm(content_h, axis=-1, keepdims=True)
        n_s_h = jnp.linalg.norm(style_h, axis=-1, keepdims=True)
        cidx3, sidx3, acc_h = a2_call(content_h, style_h, n_c_h, n_s_h,
                                      cbn_c, cbn_s)
        ci_h = cidx3.reshape(_HTOK)
        si_h = sidx3.reshape(_HTOK)
        cidx_parts.append(ci_h)
        sidx_parts.append(si_h)
        acc_parts.append(acc_h)
        y_parts.append(sc_gather(t_c_pk, t_s_pk, ci_h, si_h))

    comb_parts = [
        ln_call(y1, y2, b_comb.reshape(1, _D), ln_o_g.reshape(1, _D),
                ln_o_b.reshape(1, _D))
        for (y1, y2) in y_parts
    ]
    combined2d = jnp.concatenate(comb_parts, axis=0)
    cidx_flat = jnp.concatenate(cidx_parts)
    sidx_flat = jnp.concatenate(sidx_parts)
    acc = acc_parts[0]
    for a in acc_parts[1:]:
        acc = acc + a

    e_c = acc[0, 0]
    e_s = acc[1, 0]
    cos_sum = acc[2, 0]
    closs = 0.1 * (e_c / (_NTOK * _HALF))
    sloss = 0.1 * (e_s / (_NTOK * _HALF))
    disentangle_loss = jnp.clip(cos_sum / _NTOK, 0.0, 1.0)
    total_loss = closs + sloss + 0.5 * disentangle_loss

    combined = combined2d.reshape(B, S, D)
    cidx = cidx_flat.reshape(B, S)
    sidx = sidx_flat.reshape(B, S)
    return combined, total_loss, cidx, sidx, disentangle_loss
